# Initial kernel scaffold; baseline (speedup 1.0000x reference)
import jax
import jax.numpy as jnp
from jax.experimental import pallas as pl


def kernel(x, params):
    raise NotImplementedError("write your pallas kernel here")


# speedup vs baseline: 4.8535x; 4.8535x over previous
import jax
import jax.numpy as jnp
import numpy as np
from jax.experimental import pallas as pl

F32 = jnp.float32
I32 = jnp.int32
BIG = 1.0e4
K = 16
DM = 64


HI = jax.lax.Precision.HIGHEST
BF16 = jnp.bfloat16


def _dot(a, b):
    return jnp.dot(a.astype(BF16), b.astype(BF16), preferred_element_type=F32)


def _dot_exact(a, b):
    return jnp.dot(a, b, preferred_element_type=F32, precision=HI)


def _bcast_row(col, Tq):
    return jax.lax.dot_general(jnp.ones((Tq, 1), F32), col,
                               (((1,), (1,)), ((), ())),
                               preferred_element_type=F32, precision=HI)


def _pad_axis(a, axis, n, value=0.0):
    pad = n - a.shape[axis]
    if pad <= 0:
        return a
    cfg = [(0, 0)] * a.ndim
    cfg[axis] = (0, pad)
    return jnp.pad(a, cfg, constant_values=value)


def _wt(p, in_pad=None):
    w = jnp.transpose(p['w'])
    if in_pad is not None:
        w = _pad_axis(w, 0, in_pad)
    return w


def _bias(p):
    return p['b'][None, :]


def _wt_split3(p):
    w = jnp.transpose(p['w'])
    return jnp.concatenate([_pad_axis(w[0:3], 0, 8), w[3:]], axis=0)


def _fold_bn(conv, bn):
    s = bn['gamma'] / jnp.sqrt(bn['var'] + 1e-5)
    t = (conv['b'] - bn['mean']) * s + bn['beta']
    return s[None, :], t[None, :]


def _argext_first(vals, ext, iota, axis):
    n = vals.shape[axis]
    return jnp.min(jnp.where(vals == ext, iota, n), axis=axis, keepdims=True)


def _mlp_in_body(x_ref, w1_ref, b1_ref, w2_ref, b2_ref, o_ref):
    x = x_ref[0]
    h = jax.nn.relu(_dot(x, w1_ref[...]) + b1_ref[...])
    o_ref[0] = _dot(h, w2_ref[...]) + b2_ref[...]


def _mlp_in(xp, p1, p2):
    B, Np, _ = xp.shape
    w1, b1 = _wt(p1, 8), _bias(p1)
    w2, b2 = _wt(p2), _bias(p2)
    return pl.pallas_call(
        _mlp_in_body,
        grid=(B,),
        in_specs=[
            pl.BlockSpec((1, Np, 8), lambda b: (b, 0, 0)),
            pl.BlockSpec(w1.shape, lambda b: (0, 0)),
            pl.BlockSpec(b1.shape, lambda b: (0, 0)),
            pl.BlockSpec(w2.shape, lambda b: (0, 0)),
            pl.BlockSpec(b2.shape, lambda b: (0, 0)),
        ],
        out_specs=pl.BlockSpec((1, Np, 32), lambda b: (b, 0, 0)),
        out_shape=jax.ShapeDtypeStruct((B, Np, 32), F32),
    )(xp, w1, b1, w2, b2)


def _proj_body(f_ref, wf_ref, bf_ref, wqkv_ref, o_ref):
    x1 = _dot(f_ref[0], wf_ref[...]) + bf_ref[...]
    o_ref[0] = _dot(x1, wqkv_ref[...])


def _proj(featsP, p):
    B, Np, dp = featsP.shape
    wf, bf = _wt(p['fc1']), _bias(p['fc1'])
    wqkv = jnp.concatenate([_wt(p['w_qs']), _wt(p['w_ks']), _wt(p['w_vs'])], axis=1)
    return pl.pallas_call(
        _proj_body,
        grid=(B,),
        in_specs=[
            pl.BlockSpec((1, Np, dp), lambda b: (b, 0, 0)),
            pl.BlockSpec(wf.shape, lambda b: (0, 0)),
            pl.BlockSpec(bf.shape, lambda b: (0, 0)),
            pl.BlockSpec(wqkv.shape, lambda b: (0, 0)),
        ],
        out_specs=pl.BlockSpec((1, Np, 3 * DM), lambda b: (b, 0, 0)),
        out_shape=jax.ShapeDtypeStruct((B, Np, 3 * DM), F32),
    )(featsP, wf, bf, wqkv)


def _attn_body(xq_ref, xk_ref, pre_ref, q_ref, src_ref,
               wd1_ref, bd1_ref, wd2_ref, bd2_ref,
               wg1_ref, bg1_ref, wg2_ref, bg2_ref,
               wf2_ref, bf2_ref, res_ref, attn_ref):
    Tq = xq_ref.shape[1]
    N = xk_ref.shape[1]
    xq = xq_ref[0]
    xk = xk_ref[0]

    d2 = jax.lax.dot_general(xq.astype(BF16), xk.astype(BF16),
                             (((1,), (1,)), ((), ())),
                             preferred_element_type=F32)
    qsq = jnp.sum(xq * xq, axis=-1, keepdims=True)
    ksq = _bcast_row(jnp.sum(xk * xk, axis=-1, keepdims=True), Tq)
    d = (-2.0 * d2 + qsq) + ksq

    lane = jax.lax.broadcasted_iota(I32, (Tq, N), 1)
    idx_cols = []
    dd = d
    for _ in range(K):
        mn = jnp.min(dd, axis=-1, keepdims=True)
        amn = _argext_first(dd, mn, lane, -1)
        idx_cols.append(amn)
        dd = jnp.where(lane == amn, jnp.inf, dd)
    idx = jnp.concatenate(idx_cols, axis=1)

    oh = (jax.lax.broadcasted_iota(I32, (Tq, K, N), 2)
          == idx[:, :, None]).astype(F32).reshape(Tq * K, N)
    G = _dot_exact(oh, src_ref[0])
    kk = G[:, 0:DM]
    vv = G[:, DM:2 * DM]
    kxyz = G[:, 2 * DM:2 * DM + 8].reshape(Tq, K, 8)

    pin = (xq[:, None, :] - kxyz).reshape(Tq * K, 8)
    pos = _dot(jax.nn.relu(_dot(pin, wd1_ref[...]) + bd1_ref[...]),
               wd2_ref[...]) + bd2_ref[...]

    q = q_ref[0]
    qrep = jnp.broadcast_to(q[:, None, :], (Tq, K, DM)).reshape(Tq * K, DM)
    a_in = qrep - kk + pos
    ah = _dot(jax.nn.relu(_dot(a_in, wg1_ref[...]) + bg1_ref[...]),
              wg2_ref[...]) + bg2_ref[...]
    logits = (ah * (1.0 / np.sqrt(DM))).reshape(Tq, K, DM)
    m = jnp.max(logits, axis=1, keepdims=True)
    e = jnp.exp(logits - m)
    attn = e / jnp.sum(e, axis=1, keepdims=True)
    attn_ref[0] = attn

    vp = (vv + pos).reshape(Tq, K, DM)
    res = jnp.sum(attn * vp, axis=1)
    res_ref[0] = _dot(res, wf2_ref[...]) + bf2_ref[...] + pre_ref[0]


def _attn(xyz8, featsP, qkv, p, Tq=128):
    B, Np, dp = featsP.shape
    T = Np // Tq
    q = qkv[:, :, 0:DM]
    src = jnp.concatenate([qkv[:, :, DM:2 * DM], qkv[:, :, 2 * DM:3 * DM], xyz8],
                          axis=-1)
    wd1, bd1 = _wt(p['fc_delta'][0], 8), _bias(p['fc_delta'][0])
    wd2, bd2 = _wt(p['fc_delta'][1]), _bias(p['fc_delta'][1])
    wg1, bg1 = _wt(p['fc_gamma'][0]), _bias(p['fc_gamma'][0])
    wg2, bg2 = _wt(p['fc_gamma'][1]), _bias(p['fc_gamma'][1])
    wf2, bf2 = _wt(p['fc2']), _bias(p['fc2'])
    full = lambda a: pl.BlockSpec(a.shape, lambda b, t: tuple(0 for _ in a.shape))
    res, attn = pl.pallas_call(
        _attn_body,
        grid=(B, T),
        in_specs=[
            pl.BlockSpec((1, Tq, 8), lambda b, t: (b, t, 0)),
            pl.BlockSpec((1, Np, 8), lambda b, t: (b, 0, 0)),
            pl.BlockSpec((1, Tq, dp), lambda b, t: (b, t, 0)),
            pl.BlockSpec((1, Tq, DM), lambda b, t: (b, t, 0)),
            pl.BlockSpec((1, Np, 2 * DM + 8), lambda b, t: (b, 0, 0)),
            full(wd1), full(bd1), full(wd2), full(bd2),
            full(wg1), full(bg1), full(wg2), full(bg2),
            full(wf2), full(bf2),
        ],
        out_specs=[
            pl.BlockSpec((1, Tq, dp), lambda b, t: (b, t, 0)),
            pl.BlockSpec((1, Tq, K, DM), lambda b, t: (b, t, 0, 0)),
        ],
        out_shape=[
            jax.ShapeDtypeStruct((B, Np, dp), F32),
            jax.ShapeDtypeStruct((B, Np, K, DM), F32),
        ],
    )(xyz8, xyz8, featsP, q, src,
      wd1, bd1, wd2, bd2, wg1, bg1, wg2, bg2, wf2, bf2)
    return res, attn


def _fps_body(xs_ref, ys_ref, zs_ref, o_ref):
    N, B = xs_ref.shape
    npoint = o_ref.shape[0]
    xs, ys, zs = xs_ref[...], ys_ref[...], zs_ref[...]
    rows = jax.lax.broadcasted_iota(I32, (N, B), 0)

    def body(i, st):
        dist, far = st
        o_ref[pl.ds(i, 1), :] = far
        sel = rows == far
        cx = jnp.sum(jnp.where(sel, xs, 0.0), axis=0, keepdims=True)
        cy = jnp.sum(jnp.where(sel, ys, 0.0), axis=0, keepdims=True)
        cz = jnp.sum(jnp.where(sel, zs, 0.0), axis=0, keepdims=True)
        dx = xs - cx
        dy = ys - cy
        dz = zs - cz
        d = dx * dx + dy * dy + dz * dz
        dist = jnp.minimum(dist, d)
        mx = jnp.max(dist, axis=0, keepdims=True)
        far = _argext_first(dist, mx, rows, 0).astype(I32)
        return dist, far

    dist0 = jnp.full((N, B), 1e10, F32)
    far0 = jnp.zeros((1, B), I32)
    jax.lax.fori_loop(0, npoint, body, (dist0, far0))


def _fps(xyz_r, npoint):
    B, N, _ = xyz_r.shape
    xs = jnp.transpose(xyz_r[:, :, 0])
    ys = jnp.transpose(xyz_r[:, :, 1])
    zs = jnp.transpose(xyz_r[:, :, 2])
    cent = pl.pallas_call(
        _fps_body,
        out_shape=jax.ShapeDtypeStruct((npoint, B), I32),
    )(xs, ys, zs)
    return jnp.transpose(cent)


def _sa_body(src_ref, fps_ref, w1_ref, s1_ref, t1_ref, w2_ref, s2_ref, t2_ref,
             oxyz_ref, opts_ref):
    N = src_ref.shape[1]
    npoint = oxyz_ref.shape[1]
    Cs = src_ref.shape[2]
    src = src_ref[0]
    xk = src[:, 0:8]
    Tq = min(128, npoint)
    for c in range(npoint // Tq):
        c0 = c * Tq
        idx_col = fps_ref[0, c0:c0 + Tq, 0:1].astype(I32)
        lane_q = jax.lax.broadcasted_iota(I32, (Tq, N), 1)
        ohq = (lane_q == idx_col).astype(F32)
        nx = _dot_exact(ohq, xk)
        oxyz_ref[0, c0:c0 + Tq, :] = nx

        d2 = jax.lax.dot_general(nx.astype(BF16), xk.astype(BF16),
                                 (((1,), (1,)), ((), ())),
                                 preferred_element_type=F32)
        qsq = jnp.sum(nx * nx, axis=-1, keepdims=True)
        ksq = _bcast_row(jnp.sum(xk * xk, axis=-1, keepdims=True), Tq)
        d = (-2.0 * d2 + qsq) + ksq

        idx_cols = []
        dd = d
        for _ in range(K):
            mn = jnp.min(dd, axis=-1, keepdims=True)
            amn = _argext_first(dd, mn, lane_q, -1)
            idx_cols.append(amn)
            dd = jnp.where(lane_q == amn, jnp.inf, dd)
        idx = jnp.concatenate(idx_cols, axis=1)

        oh = (jax.lax.broadcasted_iota(I32, (Tq, K, N), 2)
              == idx[:, :, None]).astype(F32).reshape(Tq * K, N)
        G = _dot_exact(oh, src)
        kxyz = G[:, 0:8].reshape(Tq, K, 8)
        kf = G[:, 8:Cs]
        norm = (kxyz - nx[:, None, :]).reshape(Tq * K, 8)

        h1 = _dot(norm, w1_ref[0:8, :]) + _dot(kf, w1_ref[8:, :])
        h1 = jax.nn.relu(h1 * s1_ref[...] + t1_ref[...])
        h2 = jax.nn.relu(_dot(h1, w2_ref[...]) * s2_ref[...] + t2_ref[...])
        C = h2.shape[-1]
        opts_ref[0, c0:c0 + Tq, :] = jnp.max(h2.reshape(Tq, K, C), axis=1)


def _sa(src, fps_lane, p, npoint):
    B, N, Cs = src.shape
    (conv1, bn1), (conv2, bn2) = p
    w1 = _wt_split3(conv1)
    s1, t1 = _fold_bn(conv1, bn1)
    w2 = _wt(conv2)
    s2, t2 = _fold_bn(conv2, bn2)
    C = w2.shape[1]
    full = lambda a: pl.BlockSpec(a.shape, lambda b: tuple(0 for _ in a.shape))
    return pl.pallas_call(
        _sa_body,
        grid=(B,),
        in_specs=[
            pl.BlockSpec((1, N, Cs), lambda b: (b, 0, 0)),
            pl.BlockSpec((1, npoint, 8), lambda b: (b, 0, 0)),
            full(w1), full(s1), full(t1), full(w2), full(s2), full(t2),
        ],
        out_specs=[
            pl.BlockSpec((1, npoint, 8), lambda b: (b, 0, 0)),
            pl.BlockSpec((1, npoint, C), lambda b: (b, 0, 0)),
        ],
        out_shape=[
            jax.ShapeDtypeStruct((B, npoint, 8), F32),
            jax.ShapeDtypeStruct((B, npoint, C), F32),
        ],
    )(src, fps_lane, w1, s1, t1, w2, s2, t2)


def _cls_body(x_ref, w1_ref, s1_ref, t1_ref, w2_ref, s2_ref, t2_ref, o_ref):
    h1 = jax.nn.relu(_dot(x_ref[...], w1_ref[...]) * s1_ref[...] + t1_ref[...])
    o_ref[...] = jax.nn.relu(_dot(h1, w2_ref[...]) * s2_ref[...] + t2_ref[...])


def _cls(xc, p):
    B, Cin = xc.shape
    (conv1, bn1), (conv2, bn2) = p
    w1 = _wt_split3(conv1)
    s1, t1 = _fold_bn(conv1, bn1)
    w2 = _wt(conv2)
    s2, t2 = _fold_bn(conv2, bn2)
    C = w2.shape[1]
    return pl.pallas_call(
        _cls_body,
        out_shape=jax.ShapeDtypeStruct((B, C), F32),
    )(xc, w1, s1, t1, w2, s2, t2)


def _xyz_pad(xyz, Np):
    p = _pad_axis(xyz, 1, Np, BIG)
    return _pad_axis(p, 2, 8, 0.0)


def _transformer(xyz_real, featsP, p):
    Np = featsP.shape[1]
    xyz8 = _xyz_pad(xyz_real, Np)
    qkv = _proj(featsP, p)
    return _attn(xyz8, featsP, qkv, p)


def kernel(x, params):
    B, N1, _ = x.shape
    xyz0 = x[:, :, 0:3]

    NP0 = 2176
    xp = _pad_axis(_pad_axis(x, 2, 8), 1, NP0)
    h = _mlp_in(xp, params['fc1'][0], params['fc1'][1])
    res0P, attn0P = _transformer(xyz0, h, params['t1'])
    points0 = res0P[:, :N1, :]
    attn0 = attn0P[:, :N1]

    feats = [(xyz0, points0)]
    attns = [attn0]

    xyz, points = xyz0, points0
    NPADS = (640, 256)
    for i in range(2):
        npoint = 2048 // 4 ** (i + 1)
        td = params['td'][i]
        cls_xyz = xyz[:, 0:1, :]
        xyz_r = xyz[:, 1:, :]
        cls_feat = points[:, 0:1, :]
        pts = points[:, 1:, :]
        dp = pts.shape[2]

        xc = jnp.concatenate([_pad_axis(cls_xyz[:, 0, :], 1, 8),
                              cls_feat[:, 0, :]], axis=1)
        hcls = _cls(xc, td['cls'])

        cent = _fps(xyz_r, npoint)
        fps_lane = jnp.broadcast_to(cent.astype(F32)[:, :, None],
                                    (B, npoint, 8))
        src = jnp.concatenate([_pad_axis(xyz_r, 2, 8), pts], axis=2)
        nxyz8, npts = _sa(src, fps_lane, td['sa'], npoint)

        xyz = jnp.concatenate([cls_xyz, nxyz8[:, :, 0:3]], axis=1)
        points = jnp.concatenate([hcls[:, None, :], npts], axis=1)

        NpI = NPADS[i]
        featsP = _pad_axis(points, 1, NpI)
        resP, attnP = _transformer(xyz, featsP, params['tf'][i])
        n = npoint + 1
        points = resP[:, :n, :]
        feats.append((xyz, points))
        attns.append(attnP[:, :n])

    return points, tuple(feats), tuple(attns)


# speedup vs baseline: 5.9407x; 1.2240x over previous
import jax
import jax.numpy as jnp
import numpy as np
from jax.experimental import pallas as pl

F32 = jnp.float32
I32 = jnp.int32
BIG = 1.0e4
K = 16
DM = 64


HI = jax.lax.Precision.HIGHEST
BF16 = jnp.bfloat16


def _dot(a, b):
    return jnp.dot(a.astype(BF16), b.astype(BF16), preferred_element_type=F32)


def _dot_exact(a, b):
    return jnp.dot(a, b, preferred_element_type=F32, precision=HI)


def _split3(a):
    hi = a.astype(BF16)
    r1 = a - hi.astype(F32)
    mid = r1.astype(BF16)
    lo = (r1 - mid.astype(F32)).astype(BF16)
    return hi, mid, lo


def _gather3(oh, sh, sm, sl):
    ob = oh.astype(BF16)
    g = jnp.dot(ob, sh, preferred_element_type=F32)
    g = g + jnp.dot(ob, sm, preferred_element_type=F32)
    return g + jnp.dot(ob, sl, preferred_element_type=F32)


def _bcast_row(col, Tq):
    return jax.lax.dot_general(jnp.ones((Tq, 1), F32), col,
                               (((1,), (1,)), ((), ())),
                               preferred_element_type=F32, precision=HI)


def _pad_axis(a, axis, n, value=0.0):
    pad = n - a.shape[axis]
    if pad <= 0:
        return a
    cfg = [(0, 0)] * a.ndim
    cfg[axis] = (0, pad)
    return jnp.pad(a, cfg, constant_values=value)


def _wt(p, in_pad=None):
    w = jnp.transpose(p['w'])
    if in_pad is not None:
        w = _pad_axis(w, 0, in_pad)
    return w


def _bias(p):
    return p['b'][None, :]


def _wt_split3(p):
    w = jnp.transpose(p['w'])
    return jnp.concatenate([_pad_axis(w[0:3], 0, 8), w[3:]], axis=0)


def _fold_bn(conv, bn):
    s = bn['gamma'] / jnp.sqrt(bn['var'] + 1e-5)
    t = (conv['b'] - bn['mean']) * s + bn['beta']
    return s[None, :], t[None, :]


def _argext_first(vals, ext, iota, axis):
    n = vals.shape[axis]
    return jnp.min(jnp.where(vals == ext, iota, n), axis=axis, keepdims=True)


def _mlp_in_body(x_ref, w1_ref, b1_ref, w2_ref, b2_ref, o_ref):
    x = x_ref[0]
    h = jax.nn.relu(_dot(x, w1_ref[...]) + b1_ref[...])
    o_ref[0] = _dot(h, w2_ref[...]) + b2_ref[...]


def _mlp_in(xp, p1, p2):
    B, Np, _ = xp.shape
    w1, b1 = _wt(p1, 8), _bias(p1)
    w2, b2 = _wt(p2), _bias(p2)
    return pl.pallas_call(
        _mlp_in_body,
        grid=(B,),
        in_specs=[
            pl.BlockSpec((1, Np, 8), lambda b: (b, 0, 0)),
            pl.BlockSpec(w1.shape, lambda b: (0, 0)),
            pl.BlockSpec(b1.shape, lambda b: (0, 0)),
            pl.BlockSpec(w2.shape, lambda b: (0, 0)),
            pl.BlockSpec(b2.shape, lambda b: (0, 0)),
        ],
        out_specs=pl.BlockSpec((1, Np, 32), lambda b: (b, 0, 0)),
        out_shape=jax.ShapeDtypeStruct((B, Np, 32), F32),
    )(xp, w1, b1, w2, b2)


def _proj_body(f_ref, wf_ref, bf_ref, wqkv_ref, o_ref):
    x1 = _dot(f_ref[0], wf_ref[...]) + bf_ref[...]
    o_ref[0] = _dot(x1, wqkv_ref[...])


def _proj(featsP, p):
    B, Np, dp = featsP.shape
    wf, bf = _wt(p['fc1']), _bias(p['fc1'])
    wqkv = jnp.concatenate([_wt(p['w_qs']), _wt(p['w_ks']), _wt(p['w_vs'])], axis=1)
    return pl.pallas_call(
        _proj_body,
        grid=(B,),
        in_specs=[
            pl.BlockSpec((1, Np, dp), lambda b: (b, 0, 0)),
            pl.BlockSpec(wf.shape, lambda b: (0, 0)),
            pl.BlockSpec(bf.shape, lambda b: (0, 0)),
            pl.BlockSpec(wqkv.shape, lambda b: (0, 0)),
        ],
        out_specs=pl.BlockSpec((1, Np, 3 * DM), lambda b: (b, 0, 0)),
        out_shape=jax.ShapeDtypeStruct((B, Np, 3 * DM), F32),
    )(featsP, wf, bf, wqkv)


def _attn_body(xq_ref, xk_ref, pre_ref, q_ref, sh_ref, sm_ref, sl_ref,
               wd1_ref, bd1_ref, wd2_ref, bd2_ref,
               wg1_ref, bg1_ref, wg2_ref, bg2_ref,
               wf2_ref, bf2_ref, res_ref, attn_ref):
    Tq = xq_ref.shape[1]
    N = xk_ref.shape[1]
    xq = xq_ref[0]
    xk = xk_ref[0]

    d2 = jax.lax.dot_general(xq.astype(BF16), xk.astype(BF16),
                             (((1,), (1,)), ((), ())),
                             preferred_element_type=F32)
    qsq = jnp.sum(xq * xq, axis=-1, keepdims=True)
    ksq = _bcast_row(jnp.sum(xk * xk, axis=-1, keepdims=True), Tq)
    d = (-2.0 * d2 + qsq) + ksq

    lane = jax.lax.broadcasted_iota(I32, (Tq, N), 1)
    idx_cols = []
    dd = d
    for _ in range(K):
        mn = jnp.min(dd, axis=-1, keepdims=True)
        amn = _argext_first(dd, mn, lane, -1)
        idx_cols.append(amn)
        dd = jnp.where(lane == amn, jnp.inf, dd)
    idx = jnp.concatenate(idx_cols, axis=1)

    oh = (jax.lax.broadcasted_iota(I32, (Tq, K, N), 2)
          == idx[:, :, None]).astype(F32).reshape(Tq * K, N)
    G = _gather3(oh, sh_ref[0], sm_ref[0], sl_ref[0])
    kk = G[:, 0:DM]
    vv = G[:, DM:2 * DM]
    kxyz = G[:, 2 * DM:2 * DM + 8].reshape(Tq, K, 8)

    pin = (xq[:, None, :] - kxyz).reshape(Tq * K, 8)
    pos = _dot(jax.nn.relu(_dot(pin, wd1_ref[...]) + bd1_ref[...]),
               wd2_ref[...]) + bd2_ref[...]

    q = q_ref[0]
    qrep = jnp.broadcast_to(q[:, None, :], (Tq, K, DM)).reshape(Tq * K, DM)
    a_in = qrep - kk + pos
    ah = _dot(jax.nn.relu(_dot(a_in, wg1_ref[...]) + bg1_ref[...]),
              wg2_ref[...]) + bg2_ref[...]
    logits = (ah * (1.0 / np.sqrt(DM))).reshape(Tq, K, DM)
    m = jnp.max(logits, axis=1, keepdims=True)
    e = jnp.exp(logits - m)
    attn = e / jnp.sum(e, axis=1, keepdims=True)
    attn_ref[0] = attn

    vp = (vv + pos).reshape(Tq, K, DM)
    res = jnp.sum(attn * vp, axis=1)
    res_ref[0] = _dot(res, wf2_ref[...]) + bf2_ref[...] + pre_ref[0]


def _attn(xyz8, featsP, qkv, p, Tq=128):
    B, Np, dp = featsP.shape
    T = Np // Tq
    q = qkv[:, :, 0:DM]
    src = jnp.concatenate([qkv[:, :, DM:2 * DM], qkv[:, :, 2 * DM:3 * DM], xyz8],
                          axis=-1)
    sh, sm, sl = _split3(src)
    wd1, bd1 = _wt(p['fc_delta'][0], 8), _bias(p['fc_delta'][0])
    wd2, bd2 = _wt(p['fc_delta'][1]), _bias(p['fc_delta'][1])
    wg1, bg1 = _wt(p['fc_gamma'][0]), _bias(p['fc_gamma'][0])
    wg2, bg2 = _wt(p['fc_gamma'][1]), _bias(p['fc_gamma'][1])
    wf2, bf2 = _wt(p['fc2']), _bias(p['fc2'])
    full = lambda a: pl.BlockSpec(a.shape, lambda b, t: tuple(0 for _ in a.shape))
    res, attn = pl.pallas_call(
        _attn_body,
        grid=(B, T),
        in_specs=[
            pl.BlockSpec((1, Tq, 8), lambda b, t: (b, t, 0)),
            pl.BlockSpec((1, Np, 8), lambda b, t: (b, 0, 0)),
            pl.BlockSpec((1, Tq, dp), lambda b, t: (b, t, 0)),
            pl.BlockSpec((1, Tq, DM), lambda b, t: (b, t, 0)),
            pl.BlockSpec((1, Np, 2 * DM + 8), lambda b, t: (b, 0, 0)),
            pl.BlockSpec((1, Np, 2 * DM + 8), lambda b, t: (b, 0, 0)),
            pl.BlockSpec((1, Np, 2 * DM + 8), lambda b, t: (b, 0, 0)),
            full(wd1), full(bd1), full(wd2), full(bd2),
            full(wg1), full(bg1), full(wg2), full(bg2),
            full(wf2), full(bf2),
        ],
        out_specs=[
            pl.BlockSpec((1, Tq, dp), lambda b, t: (b, t, 0)),
            pl.BlockSpec((1, Tq, K, DM), lambda b, t: (b, t, 0, 0)),
        ],
        out_shape=[
            jax.ShapeDtypeStruct((B, Np, dp), F32),
            jax.ShapeDtypeStruct((B, Np, K, DM), F32),
        ],
    )(xyz8, xyz8, featsP, q, sh, sm, sl,
      wd1, bd1, wd2, bd2, wg1, bg1, wg2, bg2, wf2, bf2)
    return res, attn


def _fps_body(xs_ref, ys_ref, zs_ref, o_ref):
    N, B = xs_ref.shape
    npoint = o_ref.shape[0]
    xs, ys, zs = xs_ref[...], ys_ref[...], zs_ref[...]
    rows = jax.lax.broadcasted_iota(I32, (N, B), 0)

    def body(i, st):
        dist, far = st
        o_ref[pl.ds(i, 1), :] = far
        sel = rows == far
        cx = jnp.sum(jnp.where(sel, xs, 0.0), axis=0, keepdims=True)
        cy = jnp.sum(jnp.where(sel, ys, 0.0), axis=0, keepdims=True)
        cz = jnp.sum(jnp.where(sel, zs, 0.0), axis=0, keepdims=True)
        dx = xs - cx
        dy = ys - cy
        dz = zs - cz
        d = dx * dx + dy * dy + dz * dz
        dist = jnp.minimum(dist, d)
        mx = jnp.max(dist, axis=0, keepdims=True)
        far = _argext_first(dist, mx, rows, 0).astype(I32)
        return dist, far

    dist0 = jnp.full((N, B), 1e10, F32)
    far0 = jnp.zeros((1, B), I32)
    jax.lax.fori_loop(0, npoint, body, (dist0, far0))


def _fps(xyz_r, npoint):
    B, N, _ = xyz_r.shape
    xs = jnp.transpose(xyz_r[:, :, 0])
    ys = jnp.transpose(xyz_r[:, :, 1])
    zs = jnp.transpose(xyz_r[:, :, 2])
    cent = pl.pallas_call(
        _fps_body,
        out_shape=jax.ShapeDtypeStruct((npoint, B), I32),
    )(xs, ys, zs)
    return jnp.transpose(cent)


def _sa_body(src_ref, fps_ref, w1_ref, s1_ref, t1_ref, w2_ref, s2_ref, t2_ref,
             oxyz_ref, opts_ref):
    N = src_ref.shape[1]
    npoint = oxyz_ref.shape[1]
    Cs = src_ref.shape[2]
    src = src_ref[0]
    xk = src[:, 0:8]
    Tq = min(128, npoint)
    for c in range(npoint // Tq):
        c0 = c * Tq
        idx_col = fps_ref[0, c0:c0 + Tq, 0:1].astype(I32)
        lane_q = jax.lax.broadcasted_iota(I32, (Tq, N), 1)
        ohq = (lane_q == idx_col).astype(F32)
        nx = _dot_exact(ohq, xk)
        oxyz_ref[0, c0:c0 + Tq, :] = nx

        d2 = jax.lax.dot_general(nx.astype(BF16), xk.astype(BF16),
                                 (((1,), (1,)), ((), ())),
                                 preferred_element_type=F32)
        qsq = jnp.sum(nx * nx, axis=-1, keepdims=True)
        ksq = _bcast_row(jnp.sum(xk * xk, axis=-1, keepdims=True), Tq)
        d = (-2.0 * d2 + qsq) + ksq

        idx_cols = []
        dd = d
        for _ in range(K):
            mn = jnp.min(dd, axis=-1, keepdims=True)
            amn = _argext_first(dd, mn, lane_q, -1)
            idx_cols.append(amn)
            dd = jnp.where(lane_q == amn, jnp.inf, dd)
        idx = jnp.concatenate(idx_cols, axis=1)

        oh = (jax.lax.broadcasted_iota(I32, (Tq, K, N), 2)
              == idx[:, :, None]).astype(F32).reshape(Tq * K, N)
        G = _dot_exact(oh, src)
        kxyz = G[:, 0:8].reshape(Tq, K, 8)
        kf = G[:, 8:Cs]
        norm = (kxyz - nx[:, None, :]).reshape(Tq * K, 8)

        h1 = _dot(norm, w1_ref[0:8, :]) + _dot(kf, w1_ref[8:, :])
        h1 = jax.nn.relu(h1 * s1_ref[...] + t1_ref[...])
        h2 = jax.nn.relu(_dot(h1, w2_ref[...]) * s2_ref[...] + t2_ref[...])
        C = h2.shape[-1]
        opts_ref[0, c0:c0 + Tq, :] = jnp.max(h2.reshape(Tq, K, C), axis=1)


def _sa(src, fps_lane, p, npoint):
    B, N, Cs = src.shape
    (conv1, bn1), (conv2, bn2) = p
    w1 = _wt_split3(conv1)
    s1, t1 = _fold_bn(conv1, bn1)
    w2 = _wt(conv2)
    s2, t2 = _fold_bn(conv2, bn2)
    C = w2.shape[1]
    full = lambda a: pl.BlockSpec(a.shape, lambda b: tuple(0 for _ in a.shape))
    return pl.pallas_call(
        _sa_body,
        grid=(B,),
        in_specs=[
            pl.BlockSpec((1, N, Cs), lambda b: (b, 0, 0)),
            pl.BlockSpec((1, npoint, 8), lambda b: (b, 0, 0)),
            full(w1), full(s1), full(t1), full(w2), full(s2), full(t2),
        ],
        out_specs=[
            pl.BlockSpec((1, npoint, 8), lambda b: (b, 0, 0)),
            pl.BlockSpec((1, npoint, C), lambda b: (b, 0, 0)),
        ],
        out_shape=[
            jax.ShapeDtypeStruct((B, npoint, 8), F32),
            jax.ShapeDtypeStruct((B, npoint, C), F32),
        ],
    )(src, fps_lane, w1, s1, t1, w2, s2, t2)


def _cls_body(x_ref, w1_ref, s1_ref, t1_ref, w2_ref, s2_ref, t2_ref, o_ref):
    h1 = jax.nn.relu(_dot(x_ref[...], w1_ref[...]) * s1_ref[...] + t1_ref[...])
    o_ref[...] = jax.nn.relu(_dot(h1, w2_ref[...]) * s2_ref[...] + t2_ref[...])


def _cls(xc, p):
    B, Cin = xc.shape
    (conv1, bn1), (conv2, bn2) = p
    w1 = _wt_split3(conv1)
    s1, t1 = _fold_bn(conv1, bn1)
    w2 = _wt(conv2)
    s2, t2 = _fold_bn(conv2, bn2)
    C = w2.shape[1]
    return pl.pallas_call(
        _cls_body,
        out_shape=jax.ShapeDtypeStruct((B, C), F32),
    )(xc, w1, s1, t1, w2, s2, t2)


def _xyz_pad(xyz, Np):
    p = _pad_axis(xyz, 1, Np, BIG)
    return _pad_axis(p, 2, 8, 0.0)


def _transformer(xyz_real, featsP, p):
    Np = featsP.shape[1]
    xyz8 = _xyz_pad(xyz_real, Np)
    qkv = _proj(featsP, p)
    return _attn(xyz8, featsP, qkv, p)


def kernel(x, params):
    B, N1, _ = x.shape
    xyz0 = x[:, :, 0:3]

    NP0 = 2176
    xp = _pad_axis(_pad_axis(x, 2, 8), 1, NP0)
    h = _mlp_in(xp, params['fc1'][0], params['fc1'][1])
    res0P, attn0P = _transformer(xyz0, h, params['t1'])
    points0 = res0P[:, :N1, :]
    attn0 = attn0P[:, :N1]

    feats = [(xyz0, points0)]
    attns = [attn0]

    xyz, points = xyz0, points0
    NPADS = (640, 256)
    for i in range(2):
        npoint = 2048 // 4 ** (i + 1)
        td = params['td'][i]
        cls_xyz = xyz[:, 0:1, :]
        xyz_r = xyz[:, 1:, :]
        cls_feat = points[:, 0:1, :]
        pts = points[:, 1:, :]
        dp = pts.shape[2]

        xc = jnp.concatenate([_pad_axis(cls_xyz[:, 0, :], 1, 8),
                              cls_feat[:, 0, :]], axis=1)
        hcls = _cls(xc, td['cls'])

        cent = _fps(xyz_r, npoint)
        fps_lane = jnp.broadcast_to(cent.astype(F32)[:, :, None],
                                    (B, npoint, 8))
        src = jnp.concatenate([_pad_axis(xyz_r, 2, 8), pts], axis=2)
        nxyz8, npts = _sa(src, fps_lane, td['sa'], npoint)

        xyz = jnp.concatenate([cls_xyz, nxyz8[:, :, 0:3]], axis=1)
        points = jnp.concatenate([hcls[:, None, :], npts], axis=1)

        NpI = NPADS[i]
        featsP = _pad_axis(points, 1, NpI)
        resP, attnP = _transformer(xyz, featsP, params['tf'][i])
        n = npoint + 1
        points = resP[:, :n, :]
        feats.append((xyz, points))
        attns.append(attnP[:, :n])

    return points, tuple(feats), tuple(attns)
